# baseline (speedup 1.0000x reference)
import functools

import jax
import jax.numpy as jnp
from jax import lax
from jax.experimental import pallas as pl
from jax.experimental.pallas import tpu as pltpu
from jax.experimental.pallas import tpu_sc as plsc

_B = 32
_N = 4096
_GH = 512
_GW = 512
_POOL = 8
_SLAB = 4
_STEPS = _B // _SLAB
_NV = _N // 16


def _pool_body(gt_ref, out_ref):
    g2 = jnp.reshape(gt_ref[...], (_SLAB * _GH, _GW))
    ri = lax.broadcasted_iota(jnp.int32, (_GW, 64), 0)
    ci = lax.broadcasted_iota(jnp.int32, (_GW, 64), 1)
    pmat = jnp.where(ri // _POOL == ci, 1.0, 0.0).astype(jnp.float32)
    a = lax.dot_general(g2, pmat, (((1,), (0,)), ((), ())))
    s = jnp.sum(jnp.reshape(a, (256, _POOL, 64)), axis=1)
    s2 = jnp.reshape(s, (128, 2, 64))
    qq = jnp.concatenate([s2[:, 0, :], s2[:, 1, :]], axis=1)
    out_ref[...] = jnp.reshape(qq, (_SLAB, 32, 128))


def _pool(gt):
    return pl.pallas_call(
        _pool_body,
        grid=(_STEPS,),
        in_specs=[pl.BlockSpec((_SLAB, _GH, _GW), lambda b: (b, 0, 0))],
        out_specs=pl.BlockSpec((_SLAB, 32, 128), lambda b: (b, 0, 0)),
        out_shape=jax.ShapeDtypeStruct((_B, 32, 128), jnp.float32),
        compiler_params=pltpu.CompilerParams(
            dimension_semantics=("arbitrary",)),
    )(gt)


def _scan256(hist_ref, tmp_ref, kk):
    run = jnp.zeros((16,), jnp.float32)
    fbin = jnp.zeros((16,), jnp.int32)
    fabove = jnp.zeros((16,), jnp.float32)
    found = jnp.zeros((16,), jnp.float32)
    one = jnp.ones((16,), jnp.float32)
    i15 = jnp.full((16,), 15, jnp.int32)
    for j in range(16):
        base = (15 - j) * 16
        h = hist_ref[pl.ds(base, 16)]
        rev = lax.rev(h, (0,))
        cs = plsc.cumsum(rev)
        tot = run + cs
        m = tot >= kk
        any_m = plsc.all_reduce_population_count(m) > 0
        lstar = jnp.minimum(plsc.all_reduce_ffs(m), 15)
        tmp_ref[...] = tot - rev
        above = plsc.load_gather(tmp_ref, [lstar])
        bin_sel = base + 15 - lstar
        hit = jnp.logical_and(found == 0.0, any_m)
        fbin = jnp.where(hit, bin_sel, fbin)
        fabove = jnp.where(hit, above, fabove)
        found = jnp.where(any_m, one, found)
        tmp_ref[...] = cs
        run = run + plsc.load_gather(tmp_ref, [i15])
    return fbin, fabove


_UNROLL = 8
_DL = _NV // _UNROLL


def _sc_body(conv_hbm, tran_hbm, pooled_hbm, kvec_hbm, wvec_hbm, out_hbm,
             cbuf, tbuf, gbuf, bits1, bits2, hist1, hist2, tmp16, kbuf,
             wbuf, obuf):
    wid = lax.axis_index("s") * 2 + lax.axis_index("c")
    pltpu.sync_copy(conv_hbm.at[wid], cbuf)
    pltpu.sync_copy(tran_hbm.at[wid], tbuf)
    pltpu.sync_copy(pooled_hbm.at[wid], gbuf)
    pltpu.sync_copy(kvec_hbm, kbuf)
    pltpu.sync_copy(wvec_hbm, wbuf)
    k = jnp.maximum(kbuf[...], 1).astype(jnp.float32)
    wv = wbuf[...]
    ones = jnp.ones((16,), jnp.float32)
    zf16 = jnp.zeros((16,), jnp.float32)

    def zero_hists():
        for i in range(16):
            hist1[pl.ds(i * 16, 16)] = zf16
            hist2[pl.ds(i * 16, 16)] = zf16

    zero_hists()

    @plsc.parallel_loop(0, _NV, unroll=_UNROLL)
    def _(i):
        sl = pl.ds(i * 16, 16)
        c = cbuf[sl]
        t = tbuf[sl]
        g = gbuf[sl]
        v1 = lax.bitcast_convert_type(jnp.abs(g - c), jnp.int32)
        v2 = lax.bitcast_convert_type(jnp.abs(g - t), jnp.int32)
        bits1[sl] = v1
        bits2[sl] = v2
        plsc.addupdate_scatter(hist1,
                               [lax.shift_right_logical(v1, 23)], ones)
        plsc.addupdate_scatter(hist2,
                               [lax.shift_right_logical(v2, 23)], ones)
    b1a, r1a = _scan256(hist1, tmp16, k)
    b1b, r1b = _scan256(hist2, tmp16, k)

    def refine(shift_prev, shift_bin, binmask, pref_a, pref_b):
        @plsc.parallel_loop(0, _NV, unroll=_UNROLL)
        def _(i):
            sl = pl.ds(i * 16, 16)
            v1 = bits1[sl]
            v2 = bits2[sl]
            m1 = lax.shift_right_logical(v1, shift_prev) == pref_a
            m2 = lax.shift_right_logical(v2, shift_prev) == pref_b
            i1 = lax.shift_right_logical(v1, shift_bin) & binmask
            i2 = lax.shift_right_logical(v2, shift_bin) & binmask
            plsc.addupdate_scatter(hist1, [i1], ones, mask=m1)
            plsc.addupdate_scatter(hist2, [i2], ones, mask=m2)

    zero_hists()
    refine(23, 15, 255, b1a, b1b)
    b2a, r2a = _scan256(hist1, tmp16, k - r1a)
    b2b, r2b = _scan256(hist2, tmp16, k - r1b)
    p2a = jnp.left_shift(b1a, 8) | b2a
    p2b = jnp.left_shift(b1b, 8) | b2b

    zero_hists()
    refine(15, 7, 255, p2a, p2b)
    b3a, r3a = _scan256(hist1, tmp16, k - r1a - r2a)
    b3b, r3b = _scan256(hist2, tmp16, k - r1b - r2b)
    p3a = jnp.left_shift(p2a, 8) | b3a
    p3b = jnp.left_shift(p2b, 8) | b3b

    zero_hists()
    refine(7, 0, 127, p3a, p3b)
    b4a, _ = _scan256(hist1, tmp16, k - r1a - r2a - r3a)
    b4b, _ = _scan256(hist2, tmp16, k - r1b - r2b - r3b)
    t1 = jnp.left_shift(p3a, 7) | b4a
    t2 = jnp.left_shift(p3b, 7) | b4b

    @plsc.parallel_loop(0, _NV, unroll=_UNROLL, carry=(zf16, zf16))
    def sums_out(i, carry):
        s_main, s_fb = carry
        sl = pl.ds(i * 16, 16)
        c = cbuf[sl]
        t = tbuf[sl]
        g = gbuf[sl]
        m1 = bits1[sl] >= t1
        m2 = bits2[sl] >= t2
        d_cg = c - g
        d_tg = t - g
        comb_t = wv * t + (1.0 - wv) * g
        comb_c = wv * c + (1.0 - wv) * g
        x1 = jnp.where(m1, c - comb_t, d_cg)
        x2 = jnp.where(m2, t - comb_c, d_tg)
        return (s_main + x1 * x1 + x2 * x2, s_fb + d_cg * d_cg + d_tg * d_tg)

    s_main, s_fb = sums_out
    obuf[pl.ds(0, 16)] = s_main
    obuf[pl.ds(16, 16)] = s_fb
    pltpu.sync_copy(obuf, out_hbm.at[wid])


def _sc_select(conv2, tran2, pooled2, kvec, wvec):
    mesh = plsc.VectorSubcoreMesh(core_axis_name="c", subcore_axis_name="s")
    fn = functools.partial(
        pl.kernel,
        mesh=mesh,
        compiler_params=pltpu.CompilerParams(needs_layout_passes=False),
        out_type=jax.ShapeDtypeStruct((_B, 32), jnp.float32),
        scratch_types=[
            pltpu.VMEM((_N,), jnp.float32),
            pltpu.VMEM((_N,), jnp.float32),
            pltpu.VMEM((_N,), jnp.float32),
            pltpu.VMEM((_N,), jnp.int32),
            pltpu.VMEM((_N,), jnp.int32),
            pltpu.VMEM((256,), jnp.float32),
            pltpu.VMEM((256,), jnp.float32),
            pltpu.VMEM((16,), jnp.float32),
            pltpu.VMEM((16,), jnp.int32),
            pltpu.VMEM((16,), jnp.float32),
            pltpu.VMEM((32,), jnp.float32),
        ],
    )(_sc_body)
    return fn(conv2, tran2, pooled2, kvec, wvec)


def kernel(dmap_conv, dmap_tran, gt_density, process):
    conv2 = dmap_conv.reshape(_B, _N)
    tran2 = dmap_tran.reshape(_B, _N)
    gt = gt_density.reshape(_B, _GH, _GW)
    p = process.astype(jnp.float32)
    num = jnp.floor(_N * (0.1 * p)).astype(jnp.int32)
    wgt = 1.0 * p

    pooled2 = _pool(gt).reshape(_B, _N)
    kvec = jnp.full((16,), num[0], jnp.int32)
    wvec = jnp.full((16,), wgt[0], jnp.float32)
    parts = _sc_select(conv2, tran2, pooled2, kvec, wvec)
    main = jnp.sum(parts[:, :16])
    fb = jnp.sum(parts[:, 16:])
    return jnp.where(num[0] < 1, fb, main)


# speedup vs baseline: 1.0278x; 1.0278x over previous
import functools

import jax
import jax.numpy as jnp
from jax import lax
from jax.experimental import pallas as pl
from jax.experimental.pallas import tpu as pltpu
from jax.experimental.pallas import tpu_sc as plsc

_B = 32
_N = 4096
_GH = 512
_GW = 512
_POOL = 8
_SLAB = 4
_STEPS = _B // _SLAB
_NV = _N // 16


def _pool_body(gt_ref, out_ref):
    g2 = jnp.reshape(gt_ref[...], (_SLAB * _GH, _GW))
    ri = lax.broadcasted_iota(jnp.int32, (_GW, 64), 0)
    ci = lax.broadcasted_iota(jnp.int32, (_GW, 64), 1)
    pmat = jnp.where(ri // _POOL == ci, 1.0, 0.0).astype(jnp.float32)
    a = lax.dot_general(g2, pmat, (((1,), (0,)), ((), ())))
    s = jnp.sum(jnp.reshape(a, (256, _POOL, 64)), axis=1)
    s2 = jnp.reshape(s, (128, 2, 64))
    qq = jnp.concatenate([s2[:, 0, :], s2[:, 1, :]], axis=1)
    out_ref[...] = jnp.reshape(qq, (_SLAB, 32, 128))


def _pool(gt):
    return pl.pallas_call(
        _pool_body,
        grid=(_STEPS,),
        in_specs=[pl.BlockSpec((_SLAB, _GH, _GW), lambda b: (b, 0, 0))],
        out_specs=pl.BlockSpec((_SLAB, 32, 128), lambda b: (b, 0, 0)),
        out_shape=jax.ShapeDtypeStruct((_B, 32, 128), jnp.float32),
        compiler_params=pltpu.CompilerParams(
            dimension_semantics=("arbitrary",)),
    )(gt)


def _scan256(hist_ref, tmp_ref, kk):
    run = jnp.zeros((16,), jnp.float32)
    fbin = jnp.zeros((16,), jnp.int32)
    fabove = jnp.zeros((16,), jnp.float32)
    found = jnp.zeros((16,), jnp.float32)
    one = jnp.ones((16,), jnp.float32)
    i15 = jnp.full((16,), 15, jnp.int32)
    for j in range(16):
        base = (15 - j) * 16
        h = hist_ref[pl.ds(base, 16)]
        rev = lax.rev(h, (0,))
        cs = plsc.cumsum(rev)
        tot = run + cs
        m = tot >= kk
        any_m = plsc.all_reduce_population_count(m) > 0
        lstar = jnp.minimum(plsc.all_reduce_ffs(m), 15)
        tmp_ref[...] = tot - rev
        above = plsc.load_gather(tmp_ref, [lstar])
        bin_sel = base + 15 - lstar
        hit = jnp.logical_and(found == 0.0, any_m)
        fbin = jnp.where(hit, bin_sel, fbin)
        fabove = jnp.where(hit, above, fabove)
        found = jnp.where(any_m, one, found)
        tmp_ref[...] = cs
        run = run + plsc.load_gather(tmp_ref, [i15])
    return fbin, fabove


_UNROLL = 4
_DL = _NV // _UNROLL


def _sc_body(conv_hbm, tran_hbm, pooled_hbm, kvec_hbm, wvec_hbm, out_hbm,
             cbuf, tbuf, gbuf, bits1, bits2, hist1, hist2, tmp16, kbuf,
             wbuf, obuf):
    wid = lax.axis_index("s") * 2 + lax.axis_index("c")
    pltpu.sync_copy(conv_hbm.at[wid], cbuf)
    pltpu.sync_copy(tran_hbm.at[wid], tbuf)
    pltpu.sync_copy(pooled_hbm.at[wid], gbuf)
    pltpu.sync_copy(kvec_hbm, kbuf)
    pltpu.sync_copy(wvec_hbm, wbuf)
    k = jnp.maximum(kbuf[...], 1).astype(jnp.float32)
    wv = wbuf[...]
    ones = jnp.ones((16,), jnp.float32)
    zf16 = jnp.zeros((16,), jnp.float32)

    def zero_hists():
        for i in range(16):
            hist1[pl.ds(i * 16, 16)] = zf16
            hist2[pl.ds(i * 16, 16)] = zf16

    zero_hists()

    @plsc.parallel_loop(0, _NV, unroll=_UNROLL)
    def _(i):
        sl = pl.ds(i * 16, 16)
        c = cbuf[sl]
        t = tbuf[sl]
        g = gbuf[sl]
        v1 = lax.bitcast_convert_type(jnp.abs(g - c), jnp.int32)
        v2 = lax.bitcast_convert_type(jnp.abs(g - t), jnp.int32)
        bits1[sl] = v1
        bits2[sl] = v2
        plsc.addupdate_scatter(hist1,
                               [lax.shift_right_logical(v1, 23)], ones)
        plsc.addupdate_scatter(hist2,
                               [lax.shift_right_logical(v2, 23)], ones)
    b1a, r1a = _scan256(hist1, tmp16, k)
    b1b, r1b = _scan256(hist2, tmp16, k)

    def refine(shift_prev, shift_bin, binmask, pref_a, pref_b):
        @plsc.parallel_loop(0, _NV, unroll=_UNROLL)
        def _(i):
            sl = pl.ds(i * 16, 16)
            v1 = bits1[sl]
            v2 = bits2[sl]
            m1 = lax.shift_right_logical(v1, shift_prev) == pref_a
            m2 = lax.shift_right_logical(v2, shift_prev) == pref_b
            i1 = lax.shift_right_logical(v1, shift_bin) & binmask
            i2 = lax.shift_right_logical(v2, shift_bin) & binmask
            plsc.addupdate_scatter(hist1, [i1], ones, mask=m1)
            plsc.addupdate_scatter(hist2, [i2], ones, mask=m2)

    zero_hists()
    refine(23, 15, 255, b1a, b1b)
    b2a, r2a = _scan256(hist1, tmp16, k - r1a)
    b2b, r2b = _scan256(hist2, tmp16, k - r1b)
    p2a = jnp.left_shift(b1a, 8) | b2a
    p2b = jnp.left_shift(b1b, 8) | b2b

    zero_hists()
    refine(15, 7, 255, p2a, p2b)
    b3a, r3a = _scan256(hist1, tmp16, k - r1a - r2a)
    b3b, r3b = _scan256(hist2, tmp16, k - r1b - r2b)
    p3a = jnp.left_shift(p2a, 8) | b3a
    p3b = jnp.left_shift(p2b, 8) | b3b

    zero_hists()
    refine(7, 0, 127, p3a, p3b)
    b4a, _ = _scan256(hist1, tmp16, k - r1a - r2a - r3a)
    b4b, _ = _scan256(hist2, tmp16, k - r1b - r2b - r3b)
    t1 = jnp.left_shift(p3a, 7) | b4a
    t2 = jnp.left_shift(p3b, 7) | b4b

    @plsc.parallel_loop(0, _NV, unroll=_UNROLL, carry=(zf16, zf16))
    def sums_out(i, carry):
        s_main, s_fb = carry
        sl = pl.ds(i * 16, 16)
        c = cbuf[sl]
        t = tbuf[sl]
        g = gbuf[sl]
        m1 = bits1[sl] >= t1
        m2 = bits2[sl] >= t2
        d_cg = c - g
        d_tg = t - g
        comb_t = wv * t + (1.0 - wv) * g
        comb_c = wv * c + (1.0 - wv) * g
        x1 = jnp.where(m1, c - comb_t, d_cg)
        x2 = jnp.where(m2, t - comb_c, d_tg)
        return (s_main + x1 * x1 + x2 * x2, s_fb + d_cg * d_cg + d_tg * d_tg)

    s_main, s_fb = sums_out
    obuf[pl.ds(0, 16)] = s_main
    obuf[pl.ds(16, 16)] = s_fb
    pltpu.sync_copy(obuf, out_hbm.at[wid])


def _sc_select(conv2, tran2, pooled2, kvec, wvec):
    mesh = plsc.VectorSubcoreMesh(core_axis_name="c", subcore_axis_name="s")
    fn = functools.partial(
        pl.kernel,
        mesh=mesh,
        compiler_params=pltpu.CompilerParams(needs_layout_passes=False),
        out_type=jax.ShapeDtypeStruct((_B, 32), jnp.float32),
        scratch_types=[
            pltpu.VMEM((_N,), jnp.float32),
            pltpu.VMEM((_N,), jnp.float32),
            pltpu.VMEM((_N,), jnp.float32),
            pltpu.VMEM((_N,), jnp.int32),
            pltpu.VMEM((_N,), jnp.int32),
            pltpu.VMEM((256,), jnp.float32),
            pltpu.VMEM((256,), jnp.float32),
            pltpu.VMEM((16,), jnp.float32),
            pltpu.VMEM((16,), jnp.int32),
            pltpu.VMEM((16,), jnp.float32),
            pltpu.VMEM((32,), jnp.float32),
        ],
    )(_sc_body)
    return fn(conv2, tran2, pooled2, kvec, wvec)


def kernel(dmap_conv, dmap_tran, gt_density, process):
    conv2 = dmap_conv.reshape(_B, _N)
    tran2 = dmap_tran.reshape(_B, _N)
    gt = gt_density.reshape(_B, _GH, _GW)
    p = process.astype(jnp.float32)
    num = jnp.floor(_N * (0.1 * p)).astype(jnp.int32)
    wgt = 1.0 * p

    pooled2 = _pool(gt).reshape(_B, _N)
    kvec = jnp.full((16,), num[0], jnp.int32)
    wvec = jnp.full((16,), wgt[0], jnp.float32)
    parts = _sc_select(conv2, tran2, pooled2, kvec, wvec)
    main = jnp.sum(parts[:, :16])
    fb = jnp.sum(parts[:, 16:])
    return jnp.where(num[0] < 1, fb, main)
